# baseline (speedup 1.0000x reference)
import jax
import jax.numpy as jnp
from jax.experimental import pallas as pl
from jax.experimental.pallas import tpu as pltpu
from jax.experimental.pallas import tpu_sc as plsc


_GATHER_WINDOW = 128


def _sc_gather(table, flat_idx):
    n = flat_idx.shape[1]
    e = table.shape[1]
    mesh = plsc.VectorSubcoreMesh(core_axis_name="core", subcore_axis_name="subcore")

    @pl.kernel(
        out_type=jax.ShapeDtypeStruct((n, e), table.dtype),
        mesh=mesh,
    )
    def gather_kernel(tab_hbm, idx_hbm, out_hbm):
        def body(idx_vmem, out_vmem):
            pltpu.sync_copy(tab_hbm.at[idx_vmem.at[0]], out_vmem)

        pltpu.emit_pipeline(
            body,
            grid=(n // _GATHER_WINDOW,),
            in_specs=[
                pl.BlockSpec((1, _GATHER_WINDOW), index_map=lambda i: (0, i))
            ],
            out_specs=[
                pl.BlockSpec((_GATHER_WINDOW, e), index_map=lambda i: (i, 0))
            ],
            core_axis_name=("core", "subcore"),
            dimension_semantics=(pltpu.PARALLEL,),
        )(idx_hbm, out_hbm)

    return gather_kernel(table, flat_idx)


_STEPS_PER_ITER = 8


def _gru_scan(emb_tmaj, w_ih_t, w_hh_t, b):
    n_chunks, tb, e = emb_tmaj.shape
    h_dim = w_hh_t.shape[0]
    t_blk = _STEPS_PER_ITER
    l = n_chunks * t_blk

    n_strips = 2
    bs = b // n_strips

    def body(emb_ref, wih_ref, whh_ref, out_ref, hlast_ref):
        i = pl.program_id(0)

        @pl.when(i == 0)
        def _():
            hlast_ref[...] = jnp.zeros_like(hlast_ref)

        wih = wih_ref[...]
        whh = whh_ref[...]
        gi_all = jnp.dot(emb_ref[0], wih, preferred_element_type=jnp.float32)
        hs = [hlast_ref[s * bs:(s + 1) * bs, :] for s in range(n_strips)]
        for t in range(t_blk):
            for s in range(n_strips):
                h = hs[s]
                lo = t * b + s * bs
                gi = gi_all[lo:lo + bs, :]
                gh = jnp.dot(h, whh, preferred_element_type=jnp.float32)
                tr = jnp.tanh(gi[:, :h_dim] + gh[:, :h_dim])
                tz = jnp.tanh(gi[:, h_dim:2 * h_dim] + gh[:, h_dim:2 * h_dim])
                ghn = gh[:, 2 * h_dim:]
                n = jnp.tanh(gi[:, 2 * h_dim:] + ghn + tr * ghn)
                d = h - n
                h = n + 0.5 * (d + tz * d)
                hs[s] = h
                out_ref[s * bs:(s + 1) * bs, t, :] = h
        for s in range(n_strips):
            hlast_ref[s * bs:(s + 1) * bs, :] = hs[s]

    out, h_last = pl.pallas_call(
        body,
        grid=(n_chunks,),
        in_specs=[
            pl.BlockSpec((1, t_blk * b, e), lambda i: (i, 0, 0)),
            pl.BlockSpec((e, 3 * h_dim), lambda i: (0, 0)),
            pl.BlockSpec((h_dim, 3 * h_dim), lambda i: (0, 0)),
        ],
        out_specs=[
            pl.BlockSpec((b, t_blk, h_dim), lambda i: (0, i, 0)),
            pl.BlockSpec((b, h_dim), lambda i: (0, 0)),
        ],
        out_shape=[
            jax.ShapeDtypeStruct((b, l, h_dim), jnp.float32),
            jax.ShapeDtypeStruct((b, h_dim), jnp.float32),
        ],
    )(emb_tmaj, w_ih_t, w_hh_t)
    return out, h_last


def kernel(input_sequence, hidden, table, W_ih, W_hh):
    del hidden
    b, l = input_sequence.shape
    e = table.shape[1]
    t_blk = _STEPS_PER_ITER
    idx = input_sequence.astype(jnp.int32).T.reshape(1, b * l)
    emb_tmaj = _sc_gather(table, idx).reshape(l // t_blk, t_blk * b, e)
    h_dim = W_hh.shape[1]
    scale_ih = jnp.concatenate(
        [jnp.full((2 * h_dim,), 0.5, jnp.float32),
         jnp.ones((h_dim,), jnp.float32)])
    w_ih_t = W_ih.T * scale_ih[None, :]
    w_hh_t = W_hh.T * 0.5
    out, h_last = _gru_scan(emb_tmaj, w_ih_t, w_hh_t, b)
    return out, h_last[None]


# speedup vs baseline: 1.0985x; 1.0985x over previous
import jax
import jax.numpy as jnp
from jax.experimental import pallas as pl
from jax.experimental.pallas import tpu as pltpu
from jax.experimental.pallas import tpu_sc as plsc


_GATHER_WINDOW = 128


def _sc_gather(table, flat_idx):
    n = flat_idx.shape[1]
    e = table.shape[1]
    mesh = plsc.VectorSubcoreMesh(core_axis_name="core", subcore_axis_name="subcore")

    @pl.kernel(
        out_type=jax.ShapeDtypeStruct((n, e), table.dtype),
        mesh=mesh,
    )
    def gather_kernel(tab_hbm, idx_hbm, out_hbm):
        def body(idx_vmem, out_vmem):
            pltpu.sync_copy(tab_hbm.at[idx_vmem.at[0]], out_vmem)

        pltpu.emit_pipeline(
            body,
            grid=(n // _GATHER_WINDOW,),
            in_specs=[
                pl.BlockSpec((1, _GATHER_WINDOW), index_map=lambda i: (0, i))
            ],
            out_specs=[
                pl.BlockSpec((_GATHER_WINDOW, e), index_map=lambda i: (i, 0))
            ],
            core_axis_name=("core", "subcore"),
            dimension_semantics=(pltpu.PARALLEL,),
        )(idx_hbm, out_hbm)

    return gather_kernel(table, flat_idx)


_STEPS_PER_ITER = 8


def _gru_scan(emb_tmaj, w_ih_t, w_hh_t, b):
    n_chunks, tb, e = emb_tmaj.shape
    h_dim = w_hh_t.shape[0]
    t_blk = _STEPS_PER_ITER
    l = n_chunks * t_blk

    n_strips = 2
    bs = b // n_strips
    n_pos = t_blk * n_strips

    def body(emb_ref, wih_ref, whh_ref, out_ref, hlast_ref):
        i = pl.program_id(0)

        @pl.when(i == 0)
        def _():
            hlast_ref[...] = jnp.zeros_like(hlast_ref)

        wih = wih_ref[...]
        whh = whh_ref[...]
        hs = [hlast_ref[s * bs:(s + 1) * bs, :] for s in range(n_strips)]

        def issue(p):
            t, s = divmod(p, n_strips)
            x = emb_ref[0, t * b + s * bs:t * b + (s + 1) * bs, :].astype(jnp.bfloat16)
            gi = jnp.dot(x, wih, preferred_element_type=jnp.float32)
            gh = jnp.dot(hs[s].astype(jnp.bfloat16), whh,
                         preferred_element_type=jnp.float32)
            return gi, gh

        pend = issue(0)
        for p in range(n_pos):
            t, s = divmod(p, n_strips)
            gi, gh = pend
            if p + 1 < n_pos:
                pend = issue(p + 1)
            tr = jnp.tanh(gi[:, :h_dim] + gh[:, :h_dim])
            tz = jnp.tanh(gi[:, h_dim:2 * h_dim] + gh[:, h_dim:2 * h_dim])
            ghn = gh[:, 2 * h_dim:]
            n = jnp.tanh(gi[:, 2 * h_dim:] + ghn + tr * ghn)
            h = hs[s]
            d = h - n
            h = n + 0.5 * (d + tz * d)
            hs[s] = h
            out_ref[s * bs:(s + 1) * bs, t, :] = h
        for s in range(n_strips):
            hlast_ref[s * bs:(s + 1) * bs, :] = hs[s]

    out, h_last = pl.pallas_call(
        body,
        grid=(n_chunks,),
        in_specs=[
            pl.BlockSpec((1, t_blk * b, e), lambda i: (i, 0, 0)),
            pl.BlockSpec((e, 3 * h_dim), lambda i: (0, 0)),
            pl.BlockSpec((h_dim, 3 * h_dim), lambda i: (0, 0)),
        ],
        out_specs=[
            pl.BlockSpec((b, t_blk, h_dim), lambda i: (0, i, 0)),
            pl.BlockSpec((b, h_dim), lambda i: (0, 0)),
        ],
        out_shape=[
            jax.ShapeDtypeStruct((b, l, h_dim), jnp.float32),
            jax.ShapeDtypeStruct((b, h_dim), jnp.float32),
        ],
    )(emb_tmaj, w_ih_t, w_hh_t)
    return out, h_last


def kernel(input_sequence, hidden, table, W_ih, W_hh):
    del hidden
    b, l = input_sequence.shape
    e = table.shape[1]
    t_blk = _STEPS_PER_ITER
    idx = input_sequence.astype(jnp.int32).T.reshape(1, b * l)
    emb_tmaj = _sc_gather(table, idx).reshape(l // t_blk, t_blk * b, e)
    h_dim = W_hh.shape[1]
    scale_ih = jnp.concatenate(
        [jnp.full((2 * h_dim,), 0.5, jnp.float32),
         jnp.ones((h_dim,), jnp.float32)])
    w_ih_t = (W_ih.T * scale_ih[None, :]).astype(jnp.bfloat16)
    w_hh_t = (W_hh.T * 0.5).astype(jnp.bfloat16)
    out, h_last = _gru_scan(emb_tmaj, w_ih_t, w_hh_t, b)
    return out, h_last[None]


# speedup vs baseline: 1.3489x; 1.2279x over previous
import jax
import jax.numpy as jnp
from jax.experimental import pallas as pl
from jax.experimental.pallas import tpu as pltpu
from jax.experimental.pallas import tpu_sc as plsc


_GATHER_WINDOW = 128
_STEPS_PER_ITER = 8
_N_CHUNKS = 5


def _sc_gather(table, flat_idx):
    n = flat_idx.shape[1]
    e = table.shape[1]
    mesh = plsc.VectorSubcoreMesh(core_axis_name="core", subcore_axis_name="subcore")

    @pl.kernel(
        out_type=jax.ShapeDtypeStruct((n, e), table.dtype),
        mesh=mesh,
    )
    def gather_kernel(tab_hbm, idx_hbm, out_hbm):
        def body(idx_vmem, out_vmem):
            pltpu.sync_copy(tab_hbm.at[idx_vmem.at[0]], out_vmem)

        pltpu.emit_pipeline(
            body,
            grid=(n // _GATHER_WINDOW,),
            in_specs=[
                pl.BlockSpec((1, _GATHER_WINDOW), index_map=lambda i: (0, i))
            ],
            out_specs=[
                pl.BlockSpec((_GATHER_WINDOW, e), index_map=lambda i: (i, 0))
            ],
            core_axis_name=("core", "subcore"),
            dimension_semantics=(pltpu.PARALLEL,),
        )(idx_hbm, out_hbm)

    return gather_kernel(table, flat_idx)


def _gru_chunk(emb_c, w_ih_t, w_hh_t, h_in, out_prev, chunk, n_chunk_iters, l):
    _, tb, e = emb_c.shape
    h_dim = w_hh_t.shape[0]
    t_blk = _STEPS_PER_ITER
    b = tb // t_blk
    n_strips = 2
    bs = b // n_strips
    n_pos = t_blk * n_strips

    def body(emb_ref, wih_ref, whh_ref, hin_ref, *rest):
        out_ref, hlast_ref = rest[-2], rest[-1]
        i = pl.program_id(0)

        @pl.when(i == 0)
        def _():
            hlast_ref[...] = hin_ref[...]

        wih = wih_ref[...]
        whh = whh_ref[...]
        hs = [hlast_ref[s * bs:(s + 1) * bs, :] for s in range(n_strips)]

        def issue(p):
            t, s = divmod(p, n_strips)
            x = emb_ref[0, t * b + s * bs:t * b + (s + 1) * bs, :].astype(jnp.bfloat16)
            gi = jnp.dot(x, wih, preferred_element_type=jnp.float32)
            gh = jnp.dot(hs[s].astype(jnp.bfloat16), whh,
                         preferred_element_type=jnp.float32)
            return gi, gh

        pend = issue(0)
        for p in range(n_pos):
            t, s = divmod(p, n_strips)
            gi, gh = pend
            if p + 1 < n_pos:
                pend = issue(p + 1)
            tr = jnp.tanh(gi[:, :h_dim] + gh[:, :h_dim])
            tz = jnp.tanh(gi[:, h_dim:2 * h_dim] + gh[:, h_dim:2 * h_dim])
            ghn = gh[:, 2 * h_dim:]
            n = jnp.tanh(gi[:, 2 * h_dim:] + ghn + tr * ghn)
            h = hs[s]
            d = h - n
            h = n + 0.5 * (d + tz * d)
            hs[s] = h
            out_ref[s * bs:(s + 1) * bs, t, :] = h
        for s in range(n_strips):
            hlast_ref[s * bs:(s + 1) * bs, :] = hs[s]

    in_specs = [
        pl.BlockSpec((1, t_blk * b, e), lambda i: (i, 0, 0)),
        pl.BlockSpec((e, 3 * h_dim), lambda i: (0, 0)),
        pl.BlockSpec((h_dim, 3 * h_dim), lambda i: (0, 0)),
        pl.BlockSpec((b, h_dim), lambda i: (0, 0)),
    ]
    inputs = [emb_c, w_ih_t, w_hh_t, h_in]
    kwargs = {}
    if out_prev is not None:
        in_specs.append(pl.BlockSpec(memory_space=pl.ANY))
        inputs.append(out_prev)
        kwargs["input_output_aliases"] = {4: 0}

    out, h_last = pl.pallas_call(
        body,
        grid=(n_chunk_iters,),
        in_specs=in_specs,
        out_specs=[
            pl.BlockSpec((b, t_blk, h_dim),
                         lambda i: (0, chunk * n_chunk_iters + i, 0)),
            pl.BlockSpec((b, h_dim), lambda i: (0, 0)),
        ],
        out_shape=[
            jax.ShapeDtypeStruct((b, l, h_dim), jnp.float32),
            jax.ShapeDtypeStruct((b, h_dim), jnp.float32),
        ],
        **kwargs,
    )(*inputs)
    return out, h_last


def kernel(input_sequence, hidden, table, W_ih, W_hh):
    del hidden
    b, l = input_sequence.shape
    e = table.shape[1]
    h_dim = W_hh.shape[1]
    t_blk = _STEPS_PER_ITER
    steps_per_chunk = l // _N_CHUNKS
    n_chunk_iters = steps_per_chunk // t_blk

    idx_t = input_sequence.astype(jnp.int32).T.reshape(l * b)

    scale_ih = jnp.concatenate(
        [jnp.full((2 * h_dim,), 0.5, jnp.float32),
         jnp.ones((h_dim,), jnp.float32)])
    w_ih_t = (W_ih.T * scale_ih[None, :]).astype(jnp.bfloat16)
    w_hh_t = (W_hh.T * 0.5).astype(jnp.bfloat16)

    h = jnp.zeros((b, h_dim), jnp.float32)
    out = None
    for c in range(_N_CHUNKS):
        idx_c = jax.lax.dynamic_slice_in_dim(
            idx_t, c * steps_per_chunk * b, steps_per_chunk * b).reshape(1, -1)
        emb_c = _sc_gather(table, idx_c).reshape(
            n_chunk_iters, t_blk * b, e)
        out, h = _gru_chunk(emb_c, w_ih_t, w_hh_t, h, out, c, n_chunk_iters, l)
    return out, h[None]
